# baseline (speedup 1.0000x reference)
import functools

import numpy as np
import jax
import jax.numpy as jnp
from jax.experimental import pallas as pl
from jax.experimental.pallas import tpu as pltpu


def _score_kernel(base_ref, tw_ref, tb_ref, s_ref):
    x = base_ref[0]
    s = jnp.dot(tw_ref[...], x.T, preferred_element_type=jnp.float32)
    s_ref[0] = s + tb_ref[0, 0]


def _fused_kernel(base_ref, scaf_ref, w_ref, b_ref, wo_ref, bo_ref, s_ref,
                  out_ref, k_scr, v_scr, *, D, H, dh, BQ, S, eff_k, NKV):
    i = pl.program_id(1)
    HW = 2 * dh

    @pl.when(i < NKV)
    def _():
        y = scaf_ref[0]
        w = w_ref[...]
        b = b_ref[0]
        rows = pl.ds(i * BQ, BQ)
        k_scr[rows, :] = jnp.dot(y, w[D:2 * D].T,
                                 preferred_element_type=jnp.float32) + b[D:2 * D]
        v = jnp.dot(y, w[2 * D:].T,
                    preferred_element_type=jnp.float32) + b[2 * D:]
        pieces = []
        for h in range(H):
            pieces.append(v[:, h * dh:(h + 1) * dh])
            pieces.append(jnp.ones((BQ, dh), jnp.float32))
        v_scr[rows, :] = jnp.concatenate(pieces, axis=1)

    @pl.when(i >= NKV)
    def _():
        qb = i - NKV
        x = base_ref[0]
        scale = 1.0 / np.sqrt(dh)
        q = (jnp.dot(x, w_ref[:D].T, preferred_element_type=jnp.float32)
             + b_ref[0, :D]) * scale

        heads = []
        for h in range(H):
            sl = slice(h * dh, (h + 1) * dh)
            logits = jnp.dot(q[:, sl], k_scr[:, sl].T,
                             preferred_element_type=jnp.float32)
            p = jnp.exp(logits)
            denom = jnp.sum(p, axis=-1, keepdims=True)
            o = jnp.dot(p, v_scr[:, h * HW:h * HW + dh],
                        preferred_element_type=jnp.float32)
            heads.append(o / denom)
        attn = jnp.concatenate(heads, axis=1)

        proj = (jnp.dot(attn, wo_ref[...].T, preferred_element_type=jnp.float32)
                + bo_ref[0])

        s_all = s_ref[0]
        s_blk = s_ref[0, 0, pl.ds(qb * BQ, BQ)]
        col = jax.lax.broadcasted_iota(jnp.int32, (BQ, S), 1)
        row = jax.lax.broadcasted_iota(jnp.int32, (BQ, S), 0) + qb * BQ
        sb = s_blk[:, None]
        greater = (s_all > sb).astype(jnp.int32)
        eq_earlier = ((s_all == sb) & (col < row)).astype(jnp.int32)
        rank = jnp.sum(greater + eq_earlier, axis=1)
        mask = rank < eff_k
        out_ref[0] = jnp.where(mask[:, None], proj, x)


def kernel(base_hidden, scaffold_hidden, topk_w, topk_b, sparsity,
           in_proj_w, in_proj_b, out_proj_w, out_proj_b):
    B, S, D = base_hidden.shape
    H = 12
    dh = D // H
    BQ = 512
    NKV = S // BQ

    _c = np.float32(1.0) / (np.float32(1.0) + np.exp(-np.float32(0.5)))
    eff_k = max(1, min(S, int(S * float(_c))))

    in_b2 = in_proj_b.reshape(1, 3 * D)
    out_b2 = out_proj_b.reshape(1, D)
    tb2 = topk_b.reshape(1, 1)

    scores = pl.pallas_call(
        _score_kernel,
        grid=(B,),
        in_specs=[
            pl.BlockSpec((1, S, D), lambda b: (b, 0, 0)),
            pl.BlockSpec((1, D), lambda b: (0, 0)),
            pl.BlockSpec((1, 1), lambda b: (0, 0)),
        ],
        out_specs=pl.BlockSpec((1, 1, S), lambda b: (b, 0, 0)),
        out_shape=jax.ShapeDtypeStruct((B, 1, S), jnp.float32),
    )(base_hidden, topk_w, tb2)

    nkv = NKV

    def _qb_idx(b, i):
        return (b, jnp.maximum(i - nkv, 0), 0)

    def _kv_idx(b, i):
        return (b, jnp.minimum(i, nkv - 1), 0)

    out = pl.pallas_call(
        functools.partial(_fused_kernel, D=D, H=H, dh=dh, BQ=BQ, S=S,
                          eff_k=eff_k, NKV=NKV),
        grid=(B, 2 * NKV),
        in_specs=[
            pl.BlockSpec((1, BQ, D), _qb_idx),
            pl.BlockSpec((1, BQ, D), _kv_idx),
            pl.BlockSpec((3 * D, D), lambda b, i: (0, 0)),
            pl.BlockSpec((1, 3 * D), lambda b, i: (0, 0)),
            pl.BlockSpec((D, D), lambda b, i: (0, 0)),
            pl.BlockSpec((1, D), lambda b, i: (0, 0)),
            pl.BlockSpec((1, 1, S), lambda b, i: (b, 0, 0)),
        ],
        out_specs=pl.BlockSpec((1, BQ, D), _qb_idx),
        out_shape=jax.ShapeDtypeStruct((B, S, D), jnp.float32),
        scratch_shapes=[
            pltpu.VMEM((S, D), jnp.float32),
            pltpu.VMEM((S, 2 * D), jnp.float32),
        ],
    )(base_hidden, scaffold_hidden, in_proj_w, in_b2, out_proj_w, out_b2,
      scores)

    return out


# speedup vs baseline: 1.0715x; 1.0715x over previous
import functools

import numpy as np
import jax
import jax.numpy as jnp
from jax.experimental import pallas as pl
from jax.experimental.pallas import tpu as pltpu


def _score_kernel(base_ref, tw_ref, tb_ref, s_ref):
    x = base_ref[0]
    s = jnp.dot(tw_ref[...], x.T, preferred_element_type=jnp.float32)
    s_ref[0] = s + tb_ref[0, 0]


def _fused_kernel(base_ref, scaf_ref, w_ref, b_ref, wo_ref, bo_ref, s_ref,
                  out_ref, k_scr, v_scr, *, D, H, dh, BQ, S, eff_k, NKV):
    i = pl.program_id(1)
    HW = 2 * dh

    @pl.when(i < NKV)
    def _():
        y = scaf_ref[0]
        w = w_ref[...]
        b = b_ref[0]
        rows = pl.ds(i * BQ, BQ)
        k_scr[rows, :] = jnp.dot(y, w[D:2 * D].T,
                                 preferred_element_type=jnp.float32) + b[D:2 * D]
        v = jnp.dot(y, w[2 * D:].T,
                    preferred_element_type=jnp.float32) + b[2 * D:]
        pieces = []
        for h in range(H):
            pieces.append(v[:, h * dh:(h + 1) * dh])
            pieces.append(jnp.ones((BQ, dh), jnp.float32))
        v_scr[rows, :] = jnp.concatenate(pieces, axis=1)

    @pl.when(i >= NKV)
    def _():
        qb = i - NKV
        x = base_ref[0]
        scale = np.float32(np.log2(np.e) / np.sqrt(dh))
        q = (jnp.dot(x, w_ref[:D].T, preferred_element_type=jnp.float32)
             + b_ref[0, :D]) * scale

        heads = []
        for h in range(H):
            sl = slice(h * dh, (h + 1) * dh)
            logits = jnp.dot(q[:, sl], k_scr[:, sl].T,
                             preferred_element_type=jnp.float32)
            p = jnp.exp2(logits)
            o_aug = jnp.dot(p, v_scr[:, h * HW:(h + 1) * HW],
                            preferred_element_type=jnp.float32)
            heads.append(o_aug[:, :dh] / o_aug[:, dh:2 * dh])
        attn = jnp.concatenate(heads, axis=1)

        proj = (jnp.dot(attn, wo_ref[...].T, preferred_element_type=jnp.float32)
                + bo_ref[0])

        s_all = s_ref[0]
        s_blk = s_ref[0, 0, pl.ds(qb * BQ, BQ)]
        col = jax.lax.broadcasted_iota(jnp.int32, (BQ, S), 1)
        row = jax.lax.broadcasted_iota(jnp.int32, (BQ, S), 0) + qb * BQ
        sb = s_blk[:, None]
        greater = (s_all > sb).astype(jnp.int32)
        eq_earlier = ((s_all == sb) & (col < row)).astype(jnp.int32)
        rank = jnp.sum(greater + eq_earlier, axis=1)
        mask = rank < eff_k
        out_ref[0] = jnp.where(mask[:, None], proj, x)


def kernel(base_hidden, scaffold_hidden, topk_w, topk_b, sparsity,
           in_proj_w, in_proj_b, out_proj_w, out_proj_b):
    B, S, D = base_hidden.shape
    H = 12
    dh = D // H
    BQ = 512
    NKV = S // BQ

    _c = np.float32(1.0) / (np.float32(1.0) + np.exp(-np.float32(0.5)))
    eff_k = max(1, min(S, int(S * float(_c))))

    in_b2 = in_proj_b.reshape(1, 3 * D)
    out_b2 = out_proj_b.reshape(1, D)
    tb2 = topk_b.reshape(1, 1)

    scores = pl.pallas_call(
        _score_kernel,
        grid=(B,),
        in_specs=[
            pl.BlockSpec((1, S, D), lambda b: (b, 0, 0)),
            pl.BlockSpec((1, D), lambda b: (0, 0)),
            pl.BlockSpec((1, 1), lambda b: (0, 0)),
        ],
        out_specs=pl.BlockSpec((1, 1, S), lambda b: (b, 0, 0)),
        out_shape=jax.ShapeDtypeStruct((B, 1, S), jnp.float32),
    )(base_hidden, topk_w, tb2)

    nkv = NKV

    def _qb_idx(b, i):
        return (b, jnp.maximum(i - nkv, 0), 0)

    def _kv_idx(b, i):
        return (b, jnp.minimum(i, nkv - 1), 0)

    out = pl.pallas_call(
        functools.partial(_fused_kernel, D=D, H=H, dh=dh, BQ=BQ, S=S,
                          eff_k=eff_k, NKV=NKV),
        grid=(B, 2 * NKV),
        in_specs=[
            pl.BlockSpec((1, BQ, D), _qb_idx),
            pl.BlockSpec((1, BQ, D), _kv_idx),
            pl.BlockSpec((3 * D, D), lambda b, i: (0, 0)),
            pl.BlockSpec((1, 3 * D), lambda b, i: (0, 0)),
            pl.BlockSpec((D, D), lambda b, i: (0, 0)),
            pl.BlockSpec((1, D), lambda b, i: (0, 0)),
            pl.BlockSpec((1, 1, S), lambda b, i: (b, 0, 0)),
        ],
        out_specs=pl.BlockSpec((1, BQ, D), _qb_idx),
        out_shape=jax.ShapeDtypeStruct((B, S, D), jnp.float32),
        scratch_shapes=[
            pltpu.VMEM((S, D), jnp.float32),
            pltpu.VMEM((S, 2 * D), jnp.float32),
        ],
    )(base_hidden, scaffold_hidden, in_proj_w, in_b2, out_proj_w, out_b2,
      scores)

    return out


# speedup vs baseline: 1.1089x; 1.0349x over previous
import functools

import numpy as np
import jax
import jax.numpy as jnp
from jax.experimental import pallas as pl
from jax.experimental.pallas import tpu as pltpu


def _fused_kernel(base_ref, scaf_ref, w_ref, b_ref, wo_ref, bo_ref, tw_ref,
                  tb_ref, out_ref, k_scr, v_scr, s_scr,
                  *, D, H, dh, BQ, S, eff_k, NKV):
    i = pl.program_id(1)
    HW = 2 * dh

    @pl.when(i < NKV)
    def _():
        y = scaf_ref[0]
        w = w_ref[...]
        b = b_ref[0]
        rows = pl.ds(i * BQ, BQ)
        k_scr[rows, :] = jnp.dot(y, w[D:2 * D].T,
                                 preferred_element_type=jnp.float32) + b[D:2 * D]
        v = jnp.dot(y, w[2 * D:].T,
                    preferred_element_type=jnp.float32) + b[2 * D:]
        pieces = []
        for h in range(H):
            pieces.append(v[:, h * dh:(h + 1) * dh])
            pieces.append(jnp.ones((BQ, dh), jnp.float32))
        v_scr[rows, :] = jnp.concatenate(pieces, axis=1)
        xc = base_ref[0]
        s_scr[0, rows] = (jnp.dot(tw_ref[...], xc.T,
                                  preferred_element_type=jnp.float32)
                          + tb_ref[0, 0])[0]

    @pl.when(i >= NKV)
    def _():
        qb = i - NKV
        x = base_ref[0]
        scale = np.float32(np.log2(np.e) / np.sqrt(dh))
        q = (jnp.dot(x, w_ref[:D].T, preferred_element_type=jnp.float32)
             + b_ref[0, :D]) * scale

        heads = []
        for h in range(H):
            sl = slice(h * dh, (h + 1) * dh)
            logits = jnp.dot(q[:, sl], k_scr[:, sl].T,
                             preferred_element_type=jnp.float32)
            p = jnp.exp2(logits)
            o_aug = jnp.dot(p, v_scr[:, h * HW:(h + 1) * HW],
                            preferred_element_type=jnp.float32)
            heads.append(o_aug[:, :dh] / o_aug[:, dh:2 * dh])
        attn = jnp.concatenate(heads, axis=1)

        proj = (jnp.dot(attn, wo_ref[...].T, preferred_element_type=jnp.float32)
                + bo_ref[0])

        s_all = s_scr[...]
        s_blk = s_scr[0, pl.ds(qb * BQ, BQ)]
        col = jax.lax.broadcasted_iota(jnp.int32, (BQ, S), 1)
        row = jax.lax.broadcasted_iota(jnp.int32, (BQ, S), 0) + qb * BQ
        sb = s_blk[:, None]
        greater = (s_all > sb).astype(jnp.int32)
        eq_earlier = ((s_all == sb) & (col < row)).astype(jnp.int32)
        rank = jnp.sum(greater + eq_earlier, axis=1)
        mask = rank < eff_k
        out_ref[0] = jnp.where(mask[:, None], proj, x)


def kernel(base_hidden, scaffold_hidden, topk_w, topk_b, sparsity,
           in_proj_w, in_proj_b, out_proj_w, out_proj_b):
    B, S, D = base_hidden.shape
    H = 12
    dh = D // H
    BQ = 512
    NKV = S // BQ

    _c = np.float32(1.0) / (np.float32(1.0) + np.exp(-np.float32(0.5)))
    eff_k = max(1, min(S, int(S * float(_c))))

    in_b2 = in_proj_b.reshape(1, 3 * D)
    out_b2 = out_proj_b.reshape(1, D)
    tb2 = topk_b.reshape(1, 1)

    nkv = NKV

    def _base_idx(b, i):
        return (b, jnp.where(i < nkv, i, i - nkv), 0)

    def _qb_idx(b, i):
        return (b, jnp.maximum(i - nkv, 0), 0)

    def _kv_idx(b, i):
        return (b, jnp.minimum(i, nkv - 1), 0)

    out = pl.pallas_call(
        functools.partial(_fused_kernel, D=D, H=H, dh=dh, BQ=BQ, S=S,
                          eff_k=eff_k, NKV=NKV),
        grid=(B, 2 * NKV),
        in_specs=[
            pl.BlockSpec((1, BQ, D), _base_idx),
            pl.BlockSpec((1, BQ, D), _kv_idx),
            pl.BlockSpec((3 * D, D), lambda b, i: (0, 0)),
            pl.BlockSpec((1, 3 * D), lambda b, i: (0, 0)),
            pl.BlockSpec((D, D), lambda b, i: (0, 0)),
            pl.BlockSpec((1, D), lambda b, i: (0, 0)),
            pl.BlockSpec((1, D), lambda b, i: (0, 0)),
            pl.BlockSpec((1, 1), lambda b, i: (0, 0)),
        ],
        out_specs=pl.BlockSpec((1, BQ, D), _qb_idx),
        out_shape=jax.ShapeDtypeStruct((B, S, D), jnp.float32),
        scratch_shapes=[
            pltpu.VMEM((S, D), jnp.float32),
            pltpu.VMEM((S, 2 * D), jnp.float32),
            pltpu.VMEM((1, S), jnp.float32),
        ],
    )(base_hidden, scaffold_hidden, in_proj_w, in_b2, out_proj_w, out_b2,
      topk_w, tb2)

    return out
